# Initial kernel scaffold; baseline (speedup 1.0000x reference)
import jax
import jax.numpy as jnp
from jax.experimental import pallas as pl


def kernel(x, edge_index, W, a):
    raise NotImplementedError("write your pallas kernel here")


# speedup vs baseline: 7.4497x; 7.4497x over previous
import functools

import jax
import jax.numpy as jnp
from jax import lax
from jax.experimental import pallas as pl
from jax.experimental.pallas import tpu as pltpu
from jax.experimental.pallas import tpu_sc as plsc

N = 10000
E = 320000
D = 128

NC = 2
NS = 16
L = 16
NW = NC * NS
EPT = E // NW
CH = 80
NCHUNK = EPT // CH
NP = 10240
RPT = NP // NS


def _mesh():
    return plsc.VectorSubcoreMesh(
        core_axis_name="c", subcore_axis_name="s", num_cores=NC, num_subcores=NS
    )


def _mm_body(x_ref, wt_ref, a2_ref, h_ref, s_ref):
    h = jnp.dot(x_ref[...], wt_ref[...], preferred_element_type=jnp.float32)
    h_ref[...] = h
    s_ref[...] = jnp.dot(h, a2_ref[...], preferred_element_type=jnp.float32)


def _matmul(x, wt, a2):
    blk = 2000
    grid = N // blk
    return pl.pallas_call(
        _mm_body,
        grid=(grid,),
        in_specs=[
            pl.BlockSpec((blk, D), lambda i: (i, 0)),
            pl.BlockSpec((D, D), lambda i: (0, 0)),
            pl.BlockSpec((D, 2), lambda i: (0, 0)),
        ],
        out_specs=[
            pl.BlockSpec((blk, D), lambda i: (i, 0)),
            pl.BlockSpec((blk, 2), lambda i: (i, 0)),
        ],
        out_shape=[
            jax.ShapeDtypeStruct((N, D), jnp.float32),
            jax.ShapeDtypeStruct((N, 2), jnp.float32),
        ],
    )(x, wt, a2)


def _edge_scores_body(dst_hbm, src_hbm, s0_hbm, s1_hbm, e_hbm,
                      dst_v, src_v, s0_v, s1_v, out_v):
    wid = lax.axis_index("c") * NS + lax.axis_index("s")
    base = wid * EPT
    pltpu.sync_copy(dst_hbm.at[pl.ds(base, EPT)], dst_v)
    pltpu.sync_copy(src_hbm.at[pl.ds(base, EPT)], src_v)
    pltpu.sync_copy(s0_hbm, s0_v)
    pltpu.sync_copy(s1_hbm, s1_v)

    def body(i, _):
        off = i * L
        dvec = dst_v[pl.ds(off, L)]
        svec = src_v[pl.ds(off, L)]
        e0 = plsc.load_gather(s0_v, [dvec])
        e1 = plsc.load_gather(s1_v, [svec])
        out_v[pl.ds(off, L)] = e0 + e1
        return 0

    lax.fori_loop(0, EPT // L, body, 0)
    pltpu.sync_copy(out_v, e_hbm.at[pl.ds(base, EPT)])


def _edge_scores(dst, src, s0, s1):
    f = pl.kernel(
        _edge_scores_body,
        out_type=jax.ShapeDtypeStruct((E,), jnp.float32),
        mesh=_mesh(),
        compiler_params=pltpu.CompilerParams(needs_layout_passes=False),
        scratch_types=[
            pltpu.VMEM((EPT,), jnp.int32),
            pltpu.VMEM((EPT,), jnp.int32),
            pltpu.VMEM((N,), jnp.float32),
            pltpu.VMEM((N,), jnp.float32),
            pltpu.VMEM((EPT,), jnp.float32),
        ],
    )
    return f(dst, src, s0, s1)


def _softmax_body(e_ref, o_ref):
    e = e_ref[...]
    m = jnp.max(e)
    p = jnp.exp(e - m)
    o_ref[...] = p / jnp.sum(p)


def _softmax(e2d):
    return pl.pallas_call(
        _softmax_body,
        out_shape=jax.ShapeDtypeStruct(e2d.shape, jnp.float32),
    )(e2d)


def _scatter_body(h_hbm, dst_hbm, src_hbm, att_hbm, zero_hbm, part_hbm,
                  dst_v, src_v, att_v, rows_v, acc, sem):
    cid = lax.axis_index("c")
    sid = lax.axis_index("s")
    wid = cid * NS + sid
    base = wid * EPT

    pltpu.sync_copy(zero_hbm, acc.at[pl.ds(sid * RPT, RPT)])
    plsc.subcore_barrier()

    def chunk(i, _):
        ebase = base + i * CH
        pltpu.sync_copy(dst_hbm.at[pl.ds(ebase, CH)], dst_v)
        pltpu.sync_copy(src_hbm.at[pl.ds(ebase, CH)], src_v)
        pltpu.sync_copy(att_hbm.at[pl.ds(ebase, CH)], att_v)
        pltpu.async_copy(h_hbm.at[src_v], rows_v, sem).wait()

        def scale(r, _):
            a = plsc.load_gather(att_v, [jnp.zeros((L,), jnp.int32) + r])
            for k in range(D // L):
                rows_v[r, pl.ds(k * L, L)] = rows_v[r, pl.ds(k * L, L)] * a
            return 0

        lax.fori_loop(0, CH, scale, 0)
        pltpu.sync_copy(rows_v, acc.at[dst_v], add=True)
        return 0

    lax.fori_loop(0, NCHUNK, chunk, 0)
    plsc.subcore_barrier()
    pltpu.sync_copy(acc.at[pl.ds(sid * RPT, RPT)],
                    part_hbm.at[cid, pl.ds(sid * RPT, RPT)])


def _scatter(h, dst, src, att, zero):
    f = pl.kernel(
        _scatter_body,
        out_type=jax.ShapeDtypeStruct((NC, NP, D), jnp.float32),
        mesh=_mesh(),
        compiler_params=pltpu.CompilerParams(needs_layout_passes=False),
        scratch_types=[
            pltpu.VMEM((CH,), jnp.int32),
            pltpu.VMEM((CH,), jnp.int32),
            pltpu.VMEM((CH,), jnp.float32),
            pltpu.VMEM((CH, D), jnp.float32),
            pltpu.VMEM_SHARED((NP, D), jnp.float32),
            pltpu.SemaphoreType.DMA,
        ],
    )
    return f(h, dst, src, att, zero)


def _add_body(p_ref, o_ref):
    o_ref[...] = p_ref[0] + p_ref[1]


def _add_partials(part):
    blk = 2048
    return pl.pallas_call(
        _add_body,
        grid=(NP // blk,),
        in_specs=[pl.BlockSpec((NC, blk, D), lambda i: (0, i, 0))],
        out_specs=pl.BlockSpec((blk, D), lambda i: (i, 0)),
        out_shape=jax.ShapeDtypeStruct((NP, D), jnp.float32),
    )(part)


@jax.jit
def kernel(x, edge_index, W, a):
    wt = W.T
    a2 = a.reshape(2, D).T
    dst = edge_index[0]
    src = edge_index[1]

    h, s = _matmul(x, wt, a2)
    s0 = s[:, 0]
    s1 = s[:, 1]
    e = _edge_scores(dst, src, s0, s1)
    att = _softmax(e.reshape(E // D, D)).reshape(E)
    zero = jnp.zeros((RPT, D), jnp.float32)
    part = _scatter(h, dst, src, att, zero)
    return _add_partials(part)[:N]


# speedup vs baseline: 8.4318x; 1.1318x over previous
import functools

import jax
import jax.numpy as jnp
from jax import lax
from jax.experimental import pallas as pl
from jax.experimental.pallas import tpu as pltpu
from jax.experimental.pallas import tpu_sc as plsc

N = 10000
E = 320000
D = 128

NC = 2
NS = 16
L = 16
NW = NC * NS
EPT = E // NW
CH = 80
NCHUNK = EPT // CH
NP = 10240
RPT = NP // NS


def _mesh():
    return plsc.VectorSubcoreMesh(
        core_axis_name="c", subcore_axis_name="s", num_cores=NC, num_subcores=NS
    )


def _mm_body(x_ref, wt_ref, a2_ref, h_ref, s_ref):
    h = jnp.dot(x_ref[...], wt_ref[...], preferred_element_type=jnp.float32)
    h_ref[...] = h
    s_ref[...] = jnp.dot(h, a2_ref[...], preferred_element_type=jnp.float32)


def _matmul(x, wt, a2):
    blk = 2000
    grid = N // blk
    return pl.pallas_call(
        _mm_body,
        grid=(grid,),
        in_specs=[
            pl.BlockSpec((blk, D), lambda i: (i, 0)),
            pl.BlockSpec((D, D), lambda i: (0, 0)),
            pl.BlockSpec((D, 2), lambda i: (0, 0)),
        ],
        out_specs=[
            pl.BlockSpec((blk, D), lambda i: (i, 0)),
            pl.BlockSpec((blk, 2), lambda i: (i, 0)),
        ],
        out_shape=[
            jax.ShapeDtypeStruct((N, D), jnp.float32),
            jax.ShapeDtypeStruct((N, 2), jnp.float32),
        ],
    )(x, wt, a2)


def _edge_scores_body(dst_hbm, src_hbm, s0_hbm, s1_hbm, e_hbm,
                      dst_v, src_v, s0_v, s1_v, out_v):
    wid = lax.axis_index("c") * NS + lax.axis_index("s")
    base = wid * EPT
    pltpu.sync_copy(dst_hbm.at[pl.ds(base, EPT)], dst_v)
    pltpu.sync_copy(src_hbm.at[pl.ds(base, EPT)], src_v)
    pltpu.sync_copy(s0_hbm, s0_v)
    pltpu.sync_copy(s1_hbm, s1_v)

    def body(i, _):
        off = i * L
        dvec = dst_v[pl.ds(off, L)]
        svec = src_v[pl.ds(off, L)]
        e0 = plsc.load_gather(s0_v, [dvec])
        e1 = plsc.load_gather(s1_v, [svec])
        out_v[pl.ds(off, L)] = e0 + e1
        return 0

    lax.fori_loop(0, EPT // L, body, 0)
    pltpu.sync_copy(out_v, e_hbm.at[pl.ds(base, EPT)])


def _edge_scores(dst, src, s0, s1):
    f = pl.kernel(
        _edge_scores_body,
        out_type=jax.ShapeDtypeStruct((E,), jnp.float32),
        mesh=_mesh(),
        compiler_params=pltpu.CompilerParams(needs_layout_passes=False),
        scratch_types=[
            pltpu.VMEM((EPT,), jnp.int32),
            pltpu.VMEM((EPT,), jnp.int32),
            pltpu.VMEM((N,), jnp.float32),
            pltpu.VMEM((N,), jnp.float32),
            pltpu.VMEM((EPT,), jnp.float32),
        ],
    )
    return f(dst, src, s0, s1)


def _softmax_body(e_ref, o_ref):
    e = e_ref[...]
    m = jnp.max(e)
    p = jnp.exp(e - m)
    o_ref[...] = p / jnp.sum(p)


def _softmax(e2d):
    return pl.pallas_call(
        _softmax_body,
        out_shape=jax.ShapeDtypeStruct(e2d.shape, jnp.float32),
    )(e2d)


NBUF = 2


def _scatter_body(h_hbm, dst_hbm, src_hbm, att_hbm, zero_hbm, part_hbm,
                  attb, srcb, dstb, gbuf, sbuf, acc,
                  gsem, ssem, isem, dsem, asem):
    cid = lax.axis_index("c")
    sid = lax.axis_index("s")
    wid = cid * NS + sid
    base = wid * EPT

    pltpu.sync_copy(zero_hbm, acc.at[pl.ds(sid * RPT, RPT)])

    for b in range(NBUF):
        pltpu.sync_copy(src_hbm.at[pl.ds(base + b * CH, CH)],
                        srcb.at[pl.ds(b * CH, CH)])
        pltpu.sync_copy(att_hbm.at[pl.ds(base + b * CH, CH)],
                        attb.at[pl.ds(b * CH, CH)])
        pltpu.async_copy(h_hbm.at[srcb.at[pl.ds(b * CH, CH)]], gbuf.at[b],
                         gsem.at[b])

    plsc.subcore_barrier()

    def slot(o, b, last):
        i = o * NBUF + b
        nxt = (i + NBUF < NCHUNK) if last else None
        pltpu.make_async_copy(h_hbm.at[srcb.at[pl.ds(b * CH, CH)]],
                              gbuf.at[b], gsem.at[b]).wait()
        if not last:
            @pl.when(i + NBUF < NCHUNK)
            def _():
                pltpu.async_copy(
                    src_hbm.at[pl.ds(base + (i + NBUF) * CH, CH)],
                    srcb.at[pl.ds(b * CH, CH)], isem.at[b])
        if last:
            pltpu.make_async_copy(sbuf.at[b], acc.at[dstb.at[b]],
                                  ssem.at[b]).wait()
        else:
            @pl.when(o > 0)
            def _():
                pltpu.make_async_copy(sbuf.at[b], acc.at[dstb.at[b]],
                                      ssem.at[b]).wait()
        pltpu.async_copy(dst_hbm.at[pl.ds(base + i * CH, CH)],
                         dstb.at[b], dsem.at[b])
        if last:
            pltpu.make_async_copy(att_hbm.at[pl.ds(base + i * CH, CH)],
                                  attb.at[pl.ds(b * CH, CH)],
                                  asem.at[b]).wait()
        else:
            @pl.when(o > 0)
            def _():
                pltpu.make_async_copy(att_hbm.at[pl.ds(base + i * CH, CH)],
                                      attb.at[pl.ds(b * CH, CH)],
                                      asem.at[b]).wait()

        def scale(r, _):
            a = plsc.load_gather(attb,
                                 [jnp.zeros((L,), jnp.int32) + (b * CH + r)])
            for k in range(D // L):
                sbuf[b, r, pl.ds(k * L, L)] = gbuf[b, r, pl.ds(k * L, L)] * a
            return 0

        lax.fori_loop(0, CH, scale, 0)

        if not last:
            @pl.when(i + NBUF < NCHUNK)
            def _():
                pltpu.async_copy(
                    att_hbm.at[pl.ds(base + (i + NBUF) * CH, CH)],
                    attb.at[pl.ds(b * CH, CH)], asem.at[b])
        pltpu.make_async_copy(dst_hbm.at[pl.ds(base + i * CH, CH)],
                              dstb.at[b], dsem.at[b]).wait()
        if last:
            pltpu.sync_copy(sbuf.at[b], acc.at[dstb.at[b]], add=True)
        else:
            pltpu.async_copy(sbuf.at[b], acc.at[dstb.at[b]], ssem.at[b],
                             add=True)
            @pl.when(i + NBUF < NCHUNK)
            def _():
                pltpu.make_async_copy(
                    src_hbm.at[pl.ds(base + (i + NBUF) * CH, CH)],
                    srcb.at[pl.ds(b * CH, CH)], isem.at[b]).wait()
                pltpu.async_copy(h_hbm.at[srcb.at[pl.ds(b * CH, CH)]],
                                 gbuf.at[b], gsem.at[b])

    def outer(o, _):
        for b in range(NBUF):
            slot(o, b, last=False)
        return 0

    lax.fori_loop(0, NCHUNK // NBUF, outer, 0)
    slot(NCHUNK // NBUF, 0, last=True)
    pltpu.make_async_copy(sbuf.at[1], acc.at[dstb.at[1]], ssem.at[1]).wait()

    plsc.subcore_barrier()
    pltpu.sync_copy(acc.at[pl.ds(sid * RPT, RPT)],
                    part_hbm.at[cid, pl.ds(sid * RPT, RPT)])


def _scatter(h, dst, src, att, zero):
    f = pl.kernel(
        _scatter_body,
        out_type=jax.ShapeDtypeStruct((NC, NP, D), jnp.float32),
        mesh=_mesh(),
        compiler_params=pltpu.CompilerParams(needs_layout_passes=False),
        scratch_types=[
            pltpu.VMEM((NBUF * CH,), jnp.float32),
            pltpu.VMEM((NBUF * CH,), jnp.int32),
            pltpu.VMEM((NBUF, CH), jnp.int32),
            pltpu.VMEM((NBUF, CH, D), jnp.float32),
            pltpu.VMEM((NBUF, CH, D), jnp.float32),
            pltpu.VMEM_SHARED((NP, D), jnp.float32),
            pltpu.SemaphoreType.DMA((NBUF,)),
            pltpu.SemaphoreType.DMA((NBUF,)),
            pltpu.SemaphoreType.DMA((NBUF,)),
            pltpu.SemaphoreType.DMA((NBUF,)),
            pltpu.SemaphoreType.DMA((NBUF,)),
        ],
    )
    return f(h, dst, src, att, zero)


def _add_body(p_ref, o_ref):
    o_ref[...] = p_ref[0] + p_ref[1]


def _add_partials(part):
    blk = 2048
    return pl.pallas_call(
        _add_body,
        grid=(NP // blk,),
        in_specs=[pl.BlockSpec((NC, blk, D), lambda i: (0, i, 0))],
        out_specs=pl.BlockSpec((blk, D), lambda i: (i, 0)),
        out_shape=jax.ShapeDtypeStruct((NP, D), jnp.float32),
    )(part)


@jax.jit
def kernel(x, edge_index, W, a):
    wt = W.T
    a2 = a.reshape(2, D).T
    dst = edge_index[0]
    src = edge_index[1]

    h, s = _matmul(x, wt, a2)
    s0 = s[:, 0]
    s1 = s[:, 1]
    e = _edge_scores(dst, src, s0, s1)
    att = _softmax(e.reshape(E // D, D)).reshape(E)
    zero = jnp.zeros((RPT, D), jnp.float32)
    part = _scatter(h, dst, src, att, zero)
    return _add_partials(part)[:N]


# speedup vs baseline: 18.4141x; 2.1839x over previous
import functools

import jax
import jax.numpy as jnp
from jax import lax
from jax.experimental import pallas as pl
from jax.experimental.pallas import tpu as pltpu
from jax.experimental.pallas import tpu_sc as plsc

N = 10000
E = 320000
D = 128

NC = 2
NS = 16
L = 16
NW = NC * NS
EPT = E // NW
CH = 80
NCHUNK = EPT // CH
NP = 10240
RPT = NP // NS


def _mesh():
    return plsc.VectorSubcoreMesh(
        core_axis_name="c", subcore_axis_name="s", num_cores=NC, num_subcores=NS
    )


def _mm_body(x_ref, wt_ref, a2_ref, h_ref, s_ref):
    h = jnp.dot(x_ref[...], wt_ref[...], preferred_element_type=jnp.float32)
    h_ref[...] = h
    s_ref[...] = jnp.dot(h, a2_ref[...], preferred_element_type=jnp.float32)


def _matmul(x, wt, a2):
    blk = 2000
    grid = N // blk
    return pl.pallas_call(
        _mm_body,
        grid=(grid,),
        in_specs=[
            pl.BlockSpec((blk, D), lambda i: (i, 0)),
            pl.BlockSpec((D, D), lambda i: (0, 0)),
            pl.BlockSpec((D, 2), lambda i: (0, 0)),
        ],
        out_specs=[
            pl.BlockSpec((blk, D), lambda i: (i, 0)),
            pl.BlockSpec((blk, 2), lambda i: (i, 0)),
        ],
        out_shape=[
            jax.ShapeDtypeStruct((N, D), jnp.float32),
            jax.ShapeDtypeStruct((N, 2), jnp.float32),
        ],
    )(x, wt, a2)


def _edge_scores_body(dst_hbm, src_hbm, s0_hbm, s1_hbm, e_hbm,
                      dst_v, src_v, s0_v, s1_v, out_v):
    wid = lax.axis_index("c") * NS + lax.axis_index("s")
    base = wid * EPT
    pltpu.sync_copy(dst_hbm.at[pl.ds(base, EPT)], dst_v)
    pltpu.sync_copy(src_hbm.at[pl.ds(base, EPT)], src_v)
    pltpu.sync_copy(s0_hbm, s0_v)
    pltpu.sync_copy(s1_hbm, s1_v)

    def body(i, _):
        off = i * L
        dvec = dst_v[pl.ds(off, L)]
        svec = src_v[pl.ds(off, L)]
        e0 = plsc.load_gather(s0_v, [dvec])
        e1 = plsc.load_gather(s1_v, [svec])
        out_v[pl.ds(off, L)] = e0 + e1
        return 0

    lax.fori_loop(0, EPT // L, body, 0)
    pltpu.sync_copy(out_v, e_hbm.at[pl.ds(base, EPT)])


def _edge_scores(dst, src, s0, s1):
    f = pl.kernel(
        _edge_scores_body,
        out_type=jax.ShapeDtypeStruct((E,), jnp.float32),
        mesh=_mesh(),
        compiler_params=pltpu.CompilerParams(needs_layout_passes=False),
        scratch_types=[
            pltpu.VMEM((EPT,), jnp.int32),
            pltpu.VMEM((EPT,), jnp.int32),
            pltpu.VMEM((N,), jnp.float32),
            pltpu.VMEM((N,), jnp.float32),
            pltpu.VMEM((EPT,), jnp.float32),
        ],
    )
    return f(dst, src, s0, s1)


def _softmax_body(e_ref, o_ref):
    e = e_ref[...]
    m = jnp.max(e)
    p = jnp.exp(e - m)
    o_ref[...] = p / jnp.sum(p)


def _softmax(e2d):
    return pl.pallas_call(
        _softmax_body,
        out_shape=jax.ShapeDtypeStruct(e2d.shape, jnp.float32),
    )(e2d)


NBUF = 2


def _scatter_body(h_hbm, dst_hbm, src_hbm, att_hbm, zero_hbm, part_hbm,
                  attb, srcb, dstb, gbuf, sbuf, acc,
                  gsem, ssem, isem, dsem, asem):
    cid = lax.axis_index("c")
    sid = lax.axis_index("s")
    wid = cid * NS + sid
    base = wid * EPT

    pltpu.sync_copy(zero_hbm, acc.at[pl.ds(sid * RPT, RPT)])

    for b in range(NBUF):
        pltpu.sync_copy(src_hbm.at[pl.ds(base + b * CH, CH)],
                        srcb.at[pl.ds(b * CH, CH)])
        pltpu.sync_copy(att_hbm.at[pl.ds(base + b * CH, CH)],
                        attb.at[pl.ds(b * CH, CH)])
        pltpu.async_copy(h_hbm.at[srcb.at[pl.ds(b * CH, CH)]], gbuf.at[b],
                         gsem.at[b])

    plsc.subcore_barrier()

    def slot(o, b, last):
        i = o * NBUF + b
        nxt = (i + NBUF < NCHUNK) if last else None
        pltpu.make_async_copy(h_hbm.at[srcb.at[pl.ds(b * CH, CH)]],
                              gbuf.at[b], gsem.at[b]).wait()
        if not last:
            @pl.when(i + NBUF < NCHUNK)
            def _():
                pltpu.async_copy(
                    src_hbm.at[pl.ds(base + (i + NBUF) * CH, CH)],
                    srcb.at[pl.ds(b * CH, CH)], isem.at[b])
        if last:
            pltpu.make_async_copy(sbuf.at[b], acc.at[dstb.at[b]],
                                  ssem.at[b]).wait()
        else:
            @pl.when(o > 0)
            def _():
                pltpu.make_async_copy(sbuf.at[b], acc.at[dstb.at[b]],
                                      ssem.at[b]).wait()
        pltpu.async_copy(dst_hbm.at[pl.ds(base + i * CH, CH)],
                         dstb.at[b], dsem.at[b])
        if last:
            pltpu.make_async_copy(att_hbm.at[pl.ds(base + i * CH, CH)],
                                  attb.at[pl.ds(b * CH, CH)],
                                  asem.at[b]).wait()
        else:
            @pl.when(o > 0)
            def _():
                pltpu.make_async_copy(att_hbm.at[pl.ds(base + i * CH, CH)],
                                      attb.at[pl.ds(b * CH, CH)],
                                      asem.at[b]).wait()

        @plsc.parallel_loop(0, CH, unroll=8)
        def _(r):
            a = plsc.load_gather(attb,
                                 [jnp.zeros((L,), jnp.int32) + (b * CH + r)])
            for k in range(D // L):
                sbuf[b, r, pl.ds(k * L, L)] = gbuf[b, r, pl.ds(k * L, L)] * a

        if not last:
            @pl.when(i + NBUF < NCHUNK)
            def _():
                pltpu.async_copy(
                    att_hbm.at[pl.ds(base + (i + NBUF) * CH, CH)],
                    attb.at[pl.ds(b * CH, CH)], asem.at[b])
        pltpu.make_async_copy(dst_hbm.at[pl.ds(base + i * CH, CH)],
                              dstb.at[b], dsem.at[b]).wait()
        if last:
            pltpu.sync_copy(sbuf.at[b], acc.at[dstb.at[b]], add=True)
        else:
            pltpu.async_copy(sbuf.at[b], acc.at[dstb.at[b]], ssem.at[b],
                             add=True)
            @pl.when(i + NBUF < NCHUNK)
            def _():
                pltpu.make_async_copy(
                    src_hbm.at[pl.ds(base + (i + NBUF) * CH, CH)],
                    srcb.at[pl.ds(b * CH, CH)], isem.at[b]).wait()
                pltpu.async_copy(h_hbm.at[srcb.at[pl.ds(b * CH, CH)]],
                                 gbuf.at[b], gsem.at[b])

    def outer(o, _):
        for b in range(NBUF):
            slot(o, b, last=False)
        return 0

    lax.fori_loop(0, NCHUNK // NBUF, outer, 0)
    slot(NCHUNK // NBUF, 0, last=True)
    pltpu.make_async_copy(sbuf.at[1], acc.at[dstb.at[1]], ssem.at[1]).wait()

    plsc.subcore_barrier()
    pltpu.sync_copy(acc.at[pl.ds(sid * RPT, RPT)],
                    part_hbm.at[cid, pl.ds(sid * RPT, RPT)])


def _scatter(h, dst, src, att, zero):
    f = pl.kernel(
        _scatter_body,
        out_type=jax.ShapeDtypeStruct((NC, NP, D), jnp.float32),
        mesh=_mesh(),
        compiler_params=pltpu.CompilerParams(needs_layout_passes=False),
        scratch_types=[
            pltpu.VMEM((NBUF * CH,), jnp.float32),
            pltpu.VMEM((NBUF * CH,), jnp.int32),
            pltpu.VMEM((NBUF, CH), jnp.int32),
            pltpu.VMEM((NBUF, CH, D), jnp.float32),
            pltpu.VMEM((NBUF, CH, D), jnp.float32),
            pltpu.VMEM_SHARED((NP, D), jnp.float32),
            pltpu.SemaphoreType.DMA((NBUF,)),
            pltpu.SemaphoreType.DMA((NBUF,)),
            pltpu.SemaphoreType.DMA((NBUF,)),
            pltpu.SemaphoreType.DMA((NBUF,)),
            pltpu.SemaphoreType.DMA((NBUF,)),
        ],
    )
    return f(h, dst, src, att, zero)


def _add_body(p_ref, o_ref):
    o_ref[...] = p_ref[0] + p_ref[1]


def _add_partials(part):
    blk = 2048
    return pl.pallas_call(
        _add_body,
        grid=(NP // blk,),
        in_specs=[pl.BlockSpec((NC, blk, D), lambda i: (0, i, 0))],
        out_specs=pl.BlockSpec((blk, D), lambda i: (i, 0)),
        out_shape=jax.ShapeDtypeStruct((NP, D), jnp.float32),
    )(part)


@jax.jit
def kernel(x, edge_index, W, a):
    wt = W.T
    a2 = a.reshape(2, D).T
    dst = edge_index[0]
    src = edge_index[1]

    h, s = _matmul(x, wt, a2)
    s0 = s[:, 0]
    s1 = s[:, 1]
    e = _edge_scores(dst, src, s0, s1)
    att = _softmax(e.reshape(E // D, D)).reshape(E)
    zero = jnp.zeros((RPT, D), jnp.float32)
    part = _scatter(h, dst, src, att, zero)
    return _add_partials(part)[:N]


# speedup vs baseline: 21.2839x; 1.1559x over previous
import jax
import jax.numpy as jnp
from jax import lax
from jax.experimental import pallas as pl
from jax.experimental.pallas import tpu as pltpu
from jax.experimental.pallas import tpu_sc as plsc

N = 10000
E = 320000
D = 128

NC = 2
NS = 16
L = 16
NW = NC * NS
EPT = E // NW
CH = 80
NCHUNK = EPT // CH
NP = 10240
RPT = NP // NS
EWIN = EPT + 112


def _mesh():
    return plsc.VectorSubcoreMesh(
        core_axis_name="c", subcore_axis_name="s", num_cores=NC, num_subcores=NS
    )


_SC_PARAMS = pltpu.CompilerParams(needs_layout_passes=False)


def _mm_body(x_ref, w_ref, ar_ref, h_ref, s_ref):
    dn = (((1,), (1,)), ((), ()))
    h = lax.dot_general(x_ref[...], w_ref[...], dn,
                        preferred_element_type=jnp.float32)
    h_ref[...] = h
    s_ref[...] = lax.dot_general(h, ar_ref[...], dn,
                                 preferred_element_type=jnp.float32)


def _matmul(x, w, ar):
    blk = 2000
    return pl.pallas_call(
        _mm_body,
        grid=(N // blk,),
        in_specs=[
            pl.BlockSpec((blk, D), lambda i: (i, 0)),
            pl.BlockSpec((D, D), lambda i: (0, 0)),
            pl.BlockSpec((2, D), lambda i: (0, 0)),
        ],
        out_specs=[
            pl.BlockSpec((blk, D), lambda i: (i, 0)),
            pl.BlockSpec((blk, 2), lambda i: (i, 0)),
        ],
        out_shape=[
            jax.ShapeDtypeStruct((N, D), jnp.float32),
            jax.ShapeDtypeStruct((N, 2), jnp.float32),
        ],
    )(x, w, ar)


def _edge_scores_body(ei_hbm, s0_hbm, s1_hbm, e_hbm, dst_hbm, src_hbm,
                      eiv, s0_v, s1_v, out_v, dst_v, src_v):
    wid = lax.axis_index("c") * NS + lax.axis_index("s")
    base = wid * EPT
    base_al = pl.multiple_of(base // 128 * 128, 128)
    off = base - base_al
    pltpu.sync_copy(ei_hbm.at[:, pl.ds(base_al, EWIN)], eiv)
    pltpu.sync_copy(s0_hbm, s0_v)
    pltpu.sync_copy(s1_hbm, s1_v)

    @plsc.parallel_loop(0, EPT // L, unroll=8)
    def _(i):
        o = off + i * L
        dvec = eiv[0, pl.ds(o, L)]
        svec = eiv[1, pl.ds(o, L)]
        e0 = plsc.load_gather(s0_v, [dvec])
        e1 = plsc.load_gather(s1_v, [svec])
        out_v[pl.ds(i * L, L)] = e0 + e1
        dst_v[pl.ds(i * L, L)] = dvec
        src_v[pl.ds(i * L, L)] = svec

    pltpu.sync_copy(out_v, e_hbm.at[pl.ds(base, EPT)])
    pltpu.sync_copy(dst_v, dst_hbm.at[pl.ds(base, EPT)])
    pltpu.sync_copy(src_v, src_hbm.at[pl.ds(base, EPT)])


def _edge_scores(ei, s0, s1):
    f = pl.kernel(
        _edge_scores_body,
        out_type=(
            jax.ShapeDtypeStruct((E,), jnp.float32),
            jax.ShapeDtypeStruct((E,), jnp.int32),
            jax.ShapeDtypeStruct((E,), jnp.int32),
        ),
        mesh=_mesh(),
        compiler_params=_SC_PARAMS,
        scratch_types=[
            pltpu.VMEM((2, EWIN), jnp.int32),
            pltpu.VMEM((N,), jnp.float32),
            pltpu.VMEM((N,), jnp.float32),
            pltpu.VMEM((EPT,), jnp.float32),
            pltpu.VMEM((EPT,), jnp.int32),
            pltpu.VMEM((EPT,), jnp.int32),
        ],
    )
    return f(ei, s0, s1)


def _softmax_body(e_ref, o_ref):
    e = e_ref[...]
    m = jnp.max(e)
    p = jnp.exp(e - m)
    o_ref[...] = p / jnp.sum(p)


def _softmax(e2d):
    return pl.pallas_call(
        _softmax_body,
        out_shape=jax.ShapeDtypeStruct(e2d.shape, jnp.float32),
    )(e2d)


NBUF = 2


def _scatter_body(h_hbm, dst_hbm, src_hbm, att_hbm, zero_hbm, part_hbm,
                  attb, srcb, dstb, gbuf, sbuf, acc,
                  gsem, ssem, isem, dsem, asem):
    cid = lax.axis_index("c")
    sid = lax.axis_index("s")
    wid = cid * NS + sid
    base = wid * EPT

    pltpu.sync_copy(zero_hbm, acc.at[pl.ds(sid * RPT, RPT)])

    for b in range(NBUF):
        pltpu.sync_copy(src_hbm.at[pl.ds(base + b * CH, CH)],
                        srcb.at[pl.ds(b * CH, CH)])
        pltpu.sync_copy(att_hbm.at[pl.ds(base + b * CH, CH)],
                        attb.at[pl.ds(b * CH, CH)])
        pltpu.async_copy(h_hbm.at[srcb.at[pl.ds(b * CH, CH)]], gbuf.at[b],
                         gsem.at[b])

    plsc.subcore_barrier()

    def slot(o, b, last):
        i = o * NBUF + b
        pltpu.make_async_copy(h_hbm.at[srcb.at[pl.ds(b * CH, CH)]],
                              gbuf.at[b], gsem.at[b]).wait()
        if not last:
            @pl.when(i + NBUF < NCHUNK)
            def _():
                pltpu.async_copy(
                    src_hbm.at[pl.ds(base + (i + NBUF) * CH, CH)],
                    srcb.at[pl.ds(b * CH, CH)], isem.at[b])
        if last:
            pltpu.make_async_copy(sbuf.at[b], acc.at[dstb.at[b]],
                                  ssem.at[b]).wait()
        else:
            @pl.when(o > 0)
            def _():
                pltpu.make_async_copy(sbuf.at[b], acc.at[dstb.at[b]],
                                      ssem.at[b]).wait()
        pltpu.async_copy(dst_hbm.at[pl.ds(base + i * CH, CH)],
                         dstb.at[b], dsem.at[b])
        if last:
            pltpu.make_async_copy(att_hbm.at[pl.ds(base + i * CH, CH)],
                                  attb.at[pl.ds(b * CH, CH)],
                                  asem.at[b]).wait()
        else:
            @pl.when(o > 0)
            def _():
                pltpu.make_async_copy(att_hbm.at[pl.ds(base + i * CH, CH)],
                                      attb.at[pl.ds(b * CH, CH)],
                                      asem.at[b]).wait()

        @plsc.parallel_loop(0, CH // L, unroll=2)
        def _(g):
            av = attb[pl.ds(b * CH + g * L, L)]
            for j in range(L):
                a = lax.gather(
                    av, jnp.full((L, 1), j, jnp.int32),
                    lax.GatherDimensionNumbers(
                        offset_dims=(), collapsed_slice_dims=(0,),
                        start_index_map=(0,)),
                    slice_sizes=(1,),
                    mode=lax.GatherScatterMode.PROMISE_IN_BOUNDS)
                r = g * L + j
                for k in range(D // L):
                    sbuf[b, r, pl.ds(k * L, L)] = (
                        gbuf[b, r, pl.ds(k * L, L)] * a)

        if not last:
            @pl.when(i + NBUF < NCHUNK)
            def _():
                pltpu.async_copy(
                    att_hbm.at[pl.ds(base + (i + NBUF) * CH, CH)],
                    attb.at[pl.ds(b * CH, CH)], asem.at[b])
        pltpu.make_async_copy(dst_hbm.at[pl.ds(base + i * CH, CH)],
                              dstb.at[b], dsem.at[b]).wait()
        if last:
            pltpu.sync_copy(sbuf.at[b], acc.at[dstb.at[b]], add=True)
        else:
            pltpu.async_copy(sbuf.at[b], acc.at[dstb.at[b]], ssem.at[b],
                             add=True)
            @pl.when(i + NBUF < NCHUNK)
            def _():
                pltpu.make_async_copy(
                    src_hbm.at[pl.ds(base + (i + NBUF) * CH, CH)],
                    srcb.at[pl.ds(b * CH, CH)], isem.at[b]).wait()
                pltpu.async_copy(h_hbm.at[srcb.at[pl.ds(b * CH, CH)]],
                                 gbuf.at[b], gsem.at[b])

    def outer(o, _):
        for b in range(NBUF):
            slot(o, b, last=False)
        return 0

    lax.fori_loop(0, NCHUNK // NBUF, outer, 0)
    slot(NCHUNK // NBUF, 0, last=True)
    pltpu.make_async_copy(sbuf.at[1], acc.at[dstb.at[1]], ssem.at[1]).wait()

    plsc.subcore_barrier()
    pltpu.sync_copy(acc.at[pl.ds(sid * RPT, RPT)],
                    part_hbm.at[cid, pl.ds(sid * RPT, RPT)])


def _scatter(h, dst, src, att, zero):
    f = pl.kernel(
        _scatter_body,
        out_type=jax.ShapeDtypeStruct((NC, NP, D), jnp.float32),
        mesh=_mesh(),
        compiler_params=_SC_PARAMS,
        scratch_types=[
            pltpu.VMEM((NBUF * CH,), jnp.float32),
            pltpu.VMEM((NBUF * CH,), jnp.int32),
            pltpu.VMEM((NBUF, CH), jnp.int32),
            pltpu.VMEM((NBUF, CH, D), jnp.float32),
            pltpu.VMEM((NBUF, CH, D), jnp.float32),
            pltpu.VMEM_SHARED((NP, D), jnp.float32),
            pltpu.SemaphoreType.DMA((NBUF,)),
            pltpu.SemaphoreType.DMA((NBUF,)),
            pltpu.SemaphoreType.DMA((NBUF,)),
            pltpu.SemaphoreType.DMA((NBUF,)),
            pltpu.SemaphoreType.DMA((NBUF,)),
        ],
    )
    return f(h, dst, src, att, zero)


def _add_body(p_ref, o_ref):
    o_ref[...] = p_ref[0] + p_ref[1]


def _add_partials(part):
    blk = 2000
    return pl.pallas_call(
        _add_body,
        grid=(N // blk,),
        in_specs=[pl.BlockSpec((NC, blk, D), lambda i: (0, i, 0))],
        out_specs=pl.BlockSpec((blk, D), lambda i: (i, 0)),
        out_shape=jax.ShapeDtypeStruct((N, D), jnp.float32),
    )(part)


@jax.jit
def kernel(x, edge_index, W, a):
    ar = a.reshape(2, D)
    h, s = _matmul(x, W, ar)
    e, dst, src = _edge_scores(edge_index, s[:, 0], s[:, 1])
    att = _softmax(e.reshape(E // D, D)).reshape(E)
    zero = jnp.zeros((RPT, D), jnp.float32)
    part = _scatter(h, dst, src, att, zero)
    return _add_partials(part)


# speedup vs baseline: 22.6584x; 1.0646x over previous
import jax
import jax.numpy as jnp
from jax import lax
from jax.experimental import pallas as pl
from jax.experimental.pallas import tpu as pltpu
from jax.experimental.pallas import tpu_sc as plsc

N = 10000
E = 320000
D = 128

NC = 2
NS = 16
L = 16
NW = NC * NS
EPT = E // NW
CH = 80
NCHUNK = EPT // CH
NP = 10240
RPT = NP // NS
EWIN = EPT + 112


def _mesh():
    return plsc.VectorSubcoreMesh(
        core_axis_name="c", subcore_axis_name="s", num_cores=NC, num_subcores=NS
    )


_SC_PARAMS = pltpu.CompilerParams(needs_layout_passes=False)


def _mm_body(x_ref, w_ref, ar_ref, h_ref, s0_ref, s1_ref):
    dn = (((1,), (1,)), ((), ()))
    h = lax.dot_general(x_ref[...], w_ref[...], dn,
                        preferred_element_type=jnp.float32)
    h_ref[...] = h
    s = lax.dot_general(ar_ref[...], h, dn,
                        preferred_element_type=jnp.float32)
    s0_ref[...] = s[0]
    s1_ref[...] = s[1]


def _matmul(x, w, ar):
    return pl.pallas_call(
        _mm_body,
        out_shape=[
            jax.ShapeDtypeStruct((N, D), jnp.float32),
            jax.ShapeDtypeStruct((N,), jnp.float32),
            jax.ShapeDtypeStruct((N,), jnp.float32),
        ],
    )(x, w, ar)


def _edge_scores_body(ei_hbm, s0_hbm, s1_hbm, e_hbm, dst_hbm, src_hbm,
                      eiv, s0_v, s1_v, out_v, dst_v, src_v):
    wid = lax.axis_index("c") * NS + lax.axis_index("s")
    base = wid * EPT
    base_al = pl.multiple_of(base // 128 * 128, 128)
    off = base - base_al
    pltpu.sync_copy(ei_hbm.at[:, pl.ds(base_al, EWIN)], eiv)
    pltpu.sync_copy(s0_hbm, s0_v)
    pltpu.sync_copy(s1_hbm, s1_v)

    @plsc.parallel_loop(0, EPT // L, unroll=8)
    def _(i):
        o = off + i * L
        dvec = eiv[0, pl.ds(o, L)]
        svec = eiv[1, pl.ds(o, L)]
        e0 = plsc.load_gather(s0_v, [dvec])
        e1 = plsc.load_gather(s1_v, [svec])
        out_v[pl.ds(i * L, L)] = e0 + e1
        dst_v[pl.ds(i * L, L)] = dvec
        src_v[pl.ds(i * L, L)] = svec

    pltpu.sync_copy(out_v, e_hbm.at[pl.ds(base, EPT)])
    pltpu.sync_copy(dst_v, dst_hbm.at[pl.ds(base, EPT)])
    pltpu.sync_copy(src_v, src_hbm.at[pl.ds(base, EPT)])


def _edge_scores(ei, s0, s1):
    f = pl.kernel(
        _edge_scores_body,
        out_type=(
            jax.ShapeDtypeStruct((E,), jnp.float32),
            jax.ShapeDtypeStruct((E,), jnp.int32),
            jax.ShapeDtypeStruct((E,), jnp.int32),
        ),
        mesh=_mesh(),
        compiler_params=_SC_PARAMS,
        scratch_types=[
            pltpu.VMEM((2, EWIN), jnp.int32),
            pltpu.VMEM((N,), jnp.float32),
            pltpu.VMEM((N,), jnp.float32),
            pltpu.VMEM((EPT,), jnp.float32),
            pltpu.VMEM((EPT,), jnp.int32),
            pltpu.VMEM((EPT,), jnp.int32),
        ],
    )
    return f(ei, s0, s1)


def _softmax_body(e_ref, o_ref):
    e = e_ref[...]
    m = jnp.max(e)
    p = jnp.exp(e - m)
    o_ref[...] = p / jnp.sum(p)


def _softmax(e2d):
    return pl.pallas_call(
        _softmax_body,
        out_shape=jax.ShapeDtypeStruct(e2d.shape, jnp.float32),
    )(e2d)


NBUF = 2


def _scatter_body(h_hbm, dst_hbm, src_hbm, att_hbm, zero_hbm, part_hbm,
                  attb, srcb, dstb, gbuf, sbuf, acc,
                  gsem, ssem, isem, dsem, asem):
    cid = lax.axis_index("c")
    sid = lax.axis_index("s")
    wid = cid * NS + sid
    base = wid * EPT

    pltpu.sync_copy(zero_hbm, acc.at[pl.ds(sid * RPT, RPT)])

    for b in range(NBUF):
        pltpu.sync_copy(src_hbm.at[pl.ds(base + b * CH, CH)],
                        srcb.at[pl.ds(b * CH, CH)])
        pltpu.sync_copy(att_hbm.at[pl.ds(base + b * CH, CH)],
                        attb.at[pl.ds(b * CH, CH)])
        pltpu.async_copy(h_hbm.at[srcb.at[pl.ds(b * CH, CH)]], gbuf.at[b],
                         gsem.at[b])

    plsc.subcore_barrier()

    def slot(o, b, last):
        i = o * NBUF + b
        pltpu.make_async_copy(h_hbm.at[srcb.at[pl.ds(b * CH, CH)]],
                              gbuf.at[b], gsem.at[b]).wait()
        if not last:
            @pl.when(i + NBUF < NCHUNK)
            def _():
                pltpu.async_copy(
                    src_hbm.at[pl.ds(base + (i + NBUF) * CH, CH)],
                    srcb.at[pl.ds(b * CH, CH)], isem.at[b])
        if last:
            pltpu.make_async_copy(sbuf.at[b], acc.at[dstb.at[b]],
                                  ssem.at[b]).wait()
        else:
            @pl.when(o > 0)
            def _():
                pltpu.make_async_copy(sbuf.at[b], acc.at[dstb.at[b]],
                                      ssem.at[b]).wait()
        pltpu.async_copy(dst_hbm.at[pl.ds(base + i * CH, CH)],
                         dstb.at[b], dsem.at[b])
        if last:
            pltpu.make_async_copy(att_hbm.at[pl.ds(base + i * CH, CH)],
                                  attb.at[pl.ds(b * CH, CH)],
                                  asem.at[b]).wait()
        else:
            @pl.when(o > 0)
            def _():
                pltpu.make_async_copy(att_hbm.at[pl.ds(base + i * CH, CH)],
                                      attb.at[pl.ds(b * CH, CH)],
                                      asem.at[b]).wait()

        @plsc.parallel_loop(0, CH // L, unroll=2)
        def _(g):
            av = attb[pl.ds(b * CH + g * L, L)]
            for j in range(L):
                a = lax.gather(
                    av, jnp.full((L, 1), j, jnp.int32),
                    lax.GatherDimensionNumbers(
                        offset_dims=(), collapsed_slice_dims=(0,),
                        start_index_map=(0,)),
                    slice_sizes=(1,),
                    mode=lax.GatherScatterMode.PROMISE_IN_BOUNDS)
                r = g * L + j
                for k in range(D // L):
                    sbuf[b, r, pl.ds(k * L, L)] = (
                        gbuf[b, r, pl.ds(k * L, L)] * a)

        if not last:
            @pl.when(i + NBUF < NCHUNK)
            def _():
                pltpu.async_copy(
                    att_hbm.at[pl.ds(base + (i + NBUF) * CH, CH)],
                    attb.at[pl.ds(b * CH, CH)], asem.at[b])
        pltpu.make_async_copy(dst_hbm.at[pl.ds(base + i * CH, CH)],
                              dstb.at[b], dsem.at[b]).wait()
        if last:
            pltpu.sync_copy(sbuf.at[b], acc.at[dstb.at[b]], add=True)
        else:
            pltpu.async_copy(sbuf.at[b], acc.at[dstb.at[b]], ssem.at[b],
                             add=True)
            @pl.when(i + NBUF < NCHUNK)
            def _():
                pltpu.make_async_copy(
                    src_hbm.at[pl.ds(base + (i + NBUF) * CH, CH)],
                    srcb.at[pl.ds(b * CH, CH)], isem.at[b]).wait()
                pltpu.async_copy(h_hbm.at[srcb.at[pl.ds(b * CH, CH)]],
                                 gbuf.at[b], gsem.at[b])

    def outer(o, _):
        for b in range(NBUF):
            slot(o, b, last=False)
        return 0

    lax.fori_loop(0, NCHUNK // NBUF, outer, 0)
    slot(NCHUNK // NBUF, 0, last=True)
    pltpu.make_async_copy(sbuf.at[1], acc.at[dstb.at[1]], ssem.at[1]).wait()

    plsc.subcore_barrier()
    pltpu.sync_copy(acc.at[pl.ds(sid * RPT, RPT)],
                    part_hbm.at[cid, pl.ds(sid * RPT, RPT)])


def _scatter(h, dst, src, att, zero):
    f = pl.kernel(
        _scatter_body,
        out_type=jax.ShapeDtypeStruct((NC, NP, D), jnp.float32),
        mesh=_mesh(),
        compiler_params=_SC_PARAMS,
        scratch_types=[
            pltpu.VMEM((NBUF * CH,), jnp.float32),
            pltpu.VMEM((NBUF * CH,), jnp.int32),
            pltpu.VMEM((NBUF, CH), jnp.int32),
            pltpu.VMEM((NBUF, CH, D), jnp.float32),
            pltpu.VMEM((NBUF, CH, D), jnp.float32),
            pltpu.VMEM_SHARED((NP, D), jnp.float32),
            pltpu.SemaphoreType.DMA((NBUF,)),
            pltpu.SemaphoreType.DMA((NBUF,)),
            pltpu.SemaphoreType.DMA((NBUF,)),
            pltpu.SemaphoreType.DMA((NBUF,)),
            pltpu.SemaphoreType.DMA((NBUF,)),
        ],
    )
    return f(h, dst, src, att, zero)


def _add_body(p_ref, o_ref):
    o_ref[...] = p_ref[0] + p_ref[1]


def _add_partials(part):
    blk = 2000
    return pl.pallas_call(
        _add_body,
        grid=(N // blk,),
        in_specs=[pl.BlockSpec((NC, blk, D), lambda i: (0, i, 0))],
        out_specs=pl.BlockSpec((blk, D), lambda i: (i, 0)),
        out_shape=jax.ShapeDtypeStruct((N, D), jnp.float32),
    )(part)


@jax.jit
def kernel(x, edge_index, W, a):
    ar = a.reshape(2, D)
    h, s0, s1 = _matmul(x, W, ar)
    e, dst, src = _edge_scores(edge_index, s0, s1)
    att = _softmax(e.reshape(E // D, D)).reshape(E)
    zero = jnp.zeros((RPT, D), jnp.float32)
    part = _scatter(h, dst, src, att, zero)
    return _add_partials(part)


# speedup vs baseline: 22.6984x; 1.0018x over previous
import jax
import jax.numpy as jnp
from jax import lax
from jax.experimental import pallas as pl
from jax.experimental.pallas import tpu as pltpu
from jax.experimental.pallas import tpu_sc as plsc

N = 10000
E = 320000
D = 128

NC = 2
NS = 16
L = 16
NW = NC * NS
EPT = E // NW
CH = 80
NCHUNK = EPT // CH
NP = 10240
RPT = NP // NS
EWIN = EPT + 112


def _mesh():
    return plsc.VectorSubcoreMesh(
        core_axis_name="c", subcore_axis_name="s", num_cores=NC, num_subcores=NS
    )


_SC_PARAMS = pltpu.CompilerParams(needs_layout_passes=False)


def _mm_body(x_ref, w_ref, ar_ref, h_ref, s0_ref, s1_ref):
    dn = (((1,), (1,)), ((), ()))
    h = lax.dot_general(x_ref[...], w_ref[...], dn,
                        preferred_element_type=jnp.float32)
    h_ref[...] = h
    s = lax.dot_general(ar_ref[...], h, dn,
                        preferred_element_type=jnp.float32)
    s0_ref[...] = s[0]
    s1_ref[...] = s[1]


def _matmul(x, w, ar):
    return pl.pallas_call(
        _mm_body,
        out_shape=[
            jax.ShapeDtypeStruct((N, D), jnp.float32),
            jax.ShapeDtypeStruct((N,), jnp.float32),
            jax.ShapeDtypeStruct((N,), jnp.float32),
        ],
    )(x, w, ar)


def _edge_scores_body(ei_hbm, s0_hbm, s1_hbm, e_hbm, dst_hbm, src_hbm,
                      eiv, s0_v, s1_v, out_v, dst_v, src_v):
    wid = lax.axis_index("c") * NS + lax.axis_index("s")
    base = wid * EPT
    base_al = pl.multiple_of(base // 128 * 128, 128)
    off = base - base_al
    pltpu.sync_copy(ei_hbm.at[:, pl.ds(base_al, EWIN)], eiv)
    pltpu.sync_copy(s0_hbm, s0_v)
    pltpu.sync_copy(s1_hbm, s1_v)

    @plsc.parallel_loop(0, EPT // L, unroll=8)
    def _(i):
        o = off + i * L
        dvec = eiv[0, pl.ds(o, L)]
        svec = eiv[1, pl.ds(o, L)]
        e0 = plsc.load_gather(s0_v, [dvec])
        e1 = plsc.load_gather(s1_v, [svec])
        out_v[pl.ds(i * L, L)] = e0 + e1
        dst_v[pl.ds(i * L, L)] = dvec
        src_v[pl.ds(i * L, L)] = svec

    pltpu.sync_copy(out_v, e_hbm.at[pl.ds(base, EPT)])
    pltpu.sync_copy(dst_v, dst_hbm.at[pl.ds(base, EPT)])
    pltpu.sync_copy(src_v, src_hbm.at[pl.ds(base, EPT)])


def _edge_scores(ei, s0, s1):
    f = pl.kernel(
        _edge_scores_body,
        out_type=(
            jax.ShapeDtypeStruct((E,), jnp.float32),
            jax.ShapeDtypeStruct((E,), jnp.int32),
            jax.ShapeDtypeStruct((E,), jnp.int32),
        ),
        mesh=_mesh(),
        compiler_params=_SC_PARAMS,
        scratch_types=[
            pltpu.VMEM((2, EWIN), jnp.int32),
            pltpu.VMEM((N,), jnp.float32),
            pltpu.VMEM((N,), jnp.float32),
            pltpu.VMEM((EPT,), jnp.float32),
            pltpu.VMEM((EPT,), jnp.int32),
            pltpu.VMEM((EPT,), jnp.int32),
        ],
    )
    return f(ei, s0, s1)


def _softmax_body(e_ref, o_ref):
    e = e_ref[...]
    m = jnp.max(e)
    p = jnp.exp(e - m)
    o_ref[...] = p / jnp.sum(p)


def _softmax(e2d):
    return pl.pallas_call(
        _softmax_body,
        out_shape=jax.ShapeDtypeStruct(e2d.shape, jnp.float32),
    )(e2d)


NBUF = 2


def _scatter_body(h_hbm, dst_hbm, src_hbm, att_hbm, zero_hbm, part_hbm,
                  attb, srcb, dstb, gbuf, sbuf, acc,
                  gsem, ssem, isem, dsem, asem):
    cid = lax.axis_index("c")
    sid = lax.axis_index("s")
    wid = cid * NS + sid
    base = wid * EPT

    pltpu.sync_copy(zero_hbm, acc.at[pl.ds(sid * RPT, RPT)])

    for b in range(NBUF):
        pltpu.sync_copy(src_hbm.at[pl.ds(base + b * CH, CH)],
                        srcb.at[pl.ds(b * CH, CH)])
        pltpu.sync_copy(att_hbm.at[pl.ds(base + b * CH, CH)],
                        attb.at[pl.ds(b * CH, CH)])
        pltpu.async_copy(h_hbm.at[srcb.at[pl.ds(b * CH, CH)]], gbuf.at[b],
                         gsem.at[b])

    plsc.subcore_barrier()

    def slot(o, b, last):
        i = o * NBUF + b
        pltpu.make_async_copy(h_hbm.at[srcb.at[pl.ds(b * CH, CH)]],
                              gbuf.at[b], gsem.at[b]).wait()
        if not last:
            @pl.when(i + NBUF < NCHUNK)
            def _():
                pltpu.async_copy(
                    src_hbm.at[pl.ds(base + (i + NBUF) * CH, CH)],
                    srcb.at[pl.ds(b * CH, CH)], isem.at[b])
        if last:
            pltpu.make_async_copy(sbuf.at[b], acc.at[dstb.at[b]],
                                  ssem.at[b]).wait()
        else:
            @pl.when(o > 0)
            def _():
                pltpu.make_async_copy(sbuf.at[b], acc.at[dstb.at[b]],
                                      ssem.at[b]).wait()
        pltpu.async_copy(dst_hbm.at[pl.ds(base + i * CH, CH)],
                         dstb.at[b], dsem.at[b])
        if last:
            pltpu.make_async_copy(att_hbm.at[pl.ds(base + i * CH, CH)],
                                  attb.at[pl.ds(b * CH, CH)],
                                  asem.at[b]).wait()
        else:
            @pl.when(o > 0)
            def _():
                pltpu.make_async_copy(att_hbm.at[pl.ds(base + i * CH, CH)],
                                      attb.at[pl.ds(b * CH, CH)],
                                      asem.at[b]).wait()

        @plsc.parallel_loop(0, CH // L, unroll=5)
        def _(g):
            av = attb[pl.ds(b * CH + g * L, L)]
            for j in range(L):
                a = lax.gather(
                    av, jnp.full((L, 1), j, jnp.int32),
                    lax.GatherDimensionNumbers(
                        offset_dims=(), collapsed_slice_dims=(0,),
                        start_index_map=(0,)),
                    slice_sizes=(1,),
                    mode=lax.GatherScatterMode.PROMISE_IN_BOUNDS)
                r = g * L + j
                for k in range(D // L):
                    sbuf[b, r, pl.ds(k * L, L)] = (
                        gbuf[b, r, pl.ds(k * L, L)] * a)

        if not last:
            @pl.when(i + NBUF < NCHUNK)
            def _():
                pltpu.async_copy(
                    att_hbm.at[pl.ds(base + (i + NBUF) * CH, CH)],
                    attb.at[pl.ds(b * CH, CH)], asem.at[b])
        pltpu.make_async_copy(dst_hbm.at[pl.ds(base + i * CH, CH)],
                              dstb.at[b], dsem.at[b]).wait()
        if last:
            pltpu.sync_copy(sbuf.at[b], acc.at[dstb.at[b]], add=True)
        else:
            pltpu.async_copy(sbuf.at[b], acc.at[dstb.at[b]], ssem.at[b],
                             add=True)
            @pl.when(i + NBUF < NCHUNK)
            def _():
                pltpu.make_async_copy(
                    src_hbm.at[pl.ds(base + (i + NBUF) * CH, CH)],
                    srcb.at[pl.ds(b * CH, CH)], isem.at[b]).wait()
                pltpu.async_copy(h_hbm.at[srcb.at[pl.ds(b * CH, CH)]],
                                 gbuf.at[b], gsem.at[b])

    def outer(o, _):
        for b in range(NBUF):
            slot(o, b, last=False)
        return 0

    lax.fori_loop(0, NCHUNK // NBUF, outer, 0)
    slot(NCHUNK // NBUF, 0, last=True)
    pltpu.make_async_copy(sbuf.at[1], acc.at[dstb.at[1]], ssem.at[1]).wait()

    plsc.subcore_barrier()
    pltpu.sync_copy(acc.at[pl.ds(sid * RPT, RPT)],
                    part_hbm.at[cid, pl.ds(sid * RPT, RPT)])


def _scatter(h, dst, src, att, zero):
    f = pl.kernel(
        _scatter_body,
        out_type=jax.ShapeDtypeStruct((NC, NP, D), jnp.float32),
        mesh=_mesh(),
        compiler_params=_SC_PARAMS,
        scratch_types=[
            pltpu.VMEM((NBUF * CH,), jnp.float32),
            pltpu.VMEM((NBUF * CH,), jnp.int32),
            pltpu.VMEM((NBUF, CH), jnp.int32),
            pltpu.VMEM((NBUF, CH, D), jnp.float32),
            pltpu.VMEM((NBUF, CH, D), jnp.float32),
            pltpu.VMEM_SHARED((NP, D), jnp.float32),
            pltpu.SemaphoreType.DMA((NBUF,)),
            pltpu.SemaphoreType.DMA((NBUF,)),
            pltpu.SemaphoreType.DMA((NBUF,)),
            pltpu.SemaphoreType.DMA((NBUF,)),
            pltpu.SemaphoreType.DMA((NBUF,)),
        ],
    )
    return f(h, dst, src, att, zero)


def _add_body(p_ref, o_ref):
    o_ref[...] = p_ref[0] + p_ref[1]


def _add_partials(part):
    blk = 2000
    return pl.pallas_call(
        _add_body,
        grid=(N // blk,),
        in_specs=[pl.BlockSpec((NC, blk, D), lambda i: (0, i, 0))],
        out_specs=pl.BlockSpec((blk, D), lambda i: (i, 0)),
        out_shape=jax.ShapeDtypeStruct((N, D), jnp.float32),
    )(part)


@jax.jit
def kernel(x, edge_index, W, a):
    ar = a.reshape(2, D)
    h, s0, s1 = _matmul(x, W, ar)
    e, dst, src = _edge_scores(edge_index, s0, s1)
    att = _softmax(e.reshape(E // D, D)).reshape(E)
    zero = jnp.zeros((RPT, D), jnp.float32)
    part = _scatter(h, dst, src, att, zero)
    return _add_partials(part)
